# baseline (speedup 1.0000x reference)
import functools

import jax
import jax.numpy as jnp
from jax import lax
from jax.experimental import pallas as pl
from jax.experimental.pallas import tpu as pltpu
from jax.experimental.pallas import tpu_sc as plsc

_NC = 2
_NS = 16
_NW = _NC * _NS
_L = 16

_B = 16384
_H = 200
_D = 64
_NCLS = 2

_HA = 128
_HB = _H - _HA
_KB = 4
_GIDX = _KB * _H
_BPW = _B // _NW
_NG = _BPW // _KB
_HALF = _NG // 2

_V = 1000000
_VBW = 128
_NFB = _V // _VBW
_FPW = _NFB // _NW
_TAIL = _V - _NFB * _VBW

_mesh = plsc.VectorSubcoreMesh(core_axis_name="c", subcore_axis_name="s")


@functools.partial(
    pl.kernel,
    mesh=_mesh,
    out_type=jax.ShapeDtypeStruct((_B // 2, 2 * _D), jnp.float32),
    scratch_types=[
        pltpu.VMEM((_KB, _HA), jnp.int32),
        pltpu.VMEM((_KB, _HA), jnp.int32),
        pltpu.VMEM((_KB, _HA), jnp.int32),
        pltpu.VMEM((_KB, _HA), jnp.int32),
        pltpu.VMEM((_GIDX, _D), jnp.float32),
        pltpu.VMEM((_GIDX, _D), jnp.float32),
        pltpu.VMEM((_HALF * _KB // 2, 2 * _D), jnp.float32),
        pltpu.SemaphoreType.DMA,
        pltpu.SemaphoreType.DMA,
        pltpu.SemaphoreType.DMA,
        pltpu.SemaphoreType.DMA,
    ],
    compiler_params=pltpu.CompilerParams(use_tc_tiling_on_sc=False),
)
def _emb_pool(xa_hbm, xb_hbm, tab_hbm, out_hbm,
              idxa_0, idxa_1, idxb_0, idxb_1, rows_a, rows_b, pooled_v,
              sem_ga, sem_gb, sem_ia, sem_ib):
    wid = lax.axis_index("s") * _NC + lax.axis_index("c")
    xrow0 = wid * _BPW
    neg = jnp.full((_L,), -jnp.inf, jnp.float32)
    idxa_bufs = (idxa_0, idxa_1)
    idxb_bufs = (idxb_0, idxb_1)
    rows_bufs = (rows_a, rows_b)
    sem_g = (sem_ga, sem_gb)
    sem_i = (sem_ia, sem_ib)

    def idx_copies(g, par):
        r0 = xrow0 + g * _KB
        return [
            pltpu.make_async_copy(
                xa_hbm.at[pl.ds(r0, _KB), :], idxa_bufs[par], sem_i[par]),
            pltpu.make_async_copy(
                xb_hbm.at[pl.ds(r0, _KB), :], idxb_bufs[par], sem_i[par]),
        ]

    def gather_copies(par):
        cps = []
        for e in range(_KB):
            cps.append(pltpu.make_async_copy(
                tab_hbm.at[idxa_bufs[par].at[e, pl.ds(0, _HA)]],
                rows_bufs[par].at[pl.ds(e * _H, _HA)], sem_g[par]))
            cps.append(pltpu.make_async_copy(
                tab_hbm.at[idxb_bufs[par].at[e, pl.ds(0, _HB)]],
                rows_bufs[par].at[pl.ds(e * _H + _HA, _HB)], sem_g[par]))
        return cps

    def compute(g, par):
        rows_v = rows_bufs[par]

        def rbody(i, acc):
            nxt = []
            for e in range(_KB):
                row = e * _H + 4 * i
                ae = []
                for c in range(_D // _L):
                    v0 = rows_v[row, pl.ds(c * _L, _L)]
                    v1 = rows_v[row + 1, pl.ds(c * _L, _L)]
                    v2 = rows_v[row + 2, pl.ds(c * _L, _L)]
                    v3 = rows_v[row + 3, pl.ds(c * _L, _L)]
                    m = jnp.maximum(jnp.maximum(v0, v1), jnp.maximum(v2, v3))
                    ae.append(jnp.maximum(acc[e][c], m))
                nxt.append(tuple(ae))
            return tuple(nxt)

        acc0 = tuple(tuple(neg for _ in range(_D // _L)) for _ in range(_KB))
        acc = lax.fori_loop(0, _H // 4, rbody, acc0)
        prow = (g & (_HALF - 1)) * (_KB // 2)
        for e in range(_KB):
            for c in range(_D // _L):
                col = (e % 2) * _D + c * _L
                pooled_v[prow + e // 2, pl.ds(col, _L)] = acc[e][c]

    for cp in idx_copies(0, 0):
        cp.start()
    for cp in idx_copies(0, 0):
        cp.wait()
    for cp in gather_copies(0):
        cp.start()
    for cp in idx_copies(1, 1):
        cp.start()

    def pair(gp, carry):
        for par in (0, 1):
            g = 2 * gp + par
            nxt = 1 - par
            for cp in idx_copies(g + 1, nxt):
                cp.wait()
            for cp in gather_copies(nxt):
                cp.start()
            for cp in gather_copies(par):
                cp.wait()
            for cp in idx_copies(g + 2, par):
                cp.start()
            compute(g, par)

            @pl.when(g == _HALF - 1)
            def _():
                pltpu.sync_copy(
                    pooled_v,
                    out_hbm.at[pl.ds(wid * (_BPW // 2), _HALF * _KB // 2), :])
        return carry

    lax.fori_loop(0, (_NG - 2) // 2, pair, 0)

    for cp in gather_copies(0):
        cp.wait()
    for cp in idx_copies(_NG - 1, 1):
        cp.wait()
    for cp in gather_copies(1):
        cp.start()
    compute(_NG - 2, 0)
    for cp in gather_copies(1):
        cp.wait()
    compute(_NG - 1, 1)
    pltpu.sync_copy(
        pooled_v,
        out_hbm.at[pl.ds(wid * (_BPW // 2) + _HALF * _KB // 2,
                         _HALF * _KB // 2), :])


def _proj_body(p_ref, w_ref, b_ref, o_ref):
    o_ref[:] = (jnp.dot(p_ref[:], w_ref[:], preferred_element_type=jnp.float32)
                + b_ref[:])


_TPB = 16384


def _tpad_body(t_ref, o_ref):
    t = jnp.swapaxes(t_ref[:], 0, 1)
    o_ref[:] = jnp.concatenate(
        [t, jnp.zeros((_TPB, _D), jnp.float32)], axis=1)


def _transpose_pad(tabt):
    grid = (_V + _TPB - 1) // _TPB
    return pl.pallas_call(
        _tpad_body,
        grid=(grid,),
        in_specs=[pl.BlockSpec((_D, _TPB), lambda j: (0, j))],
        out_specs=pl.BlockSpec((_TPB, 2 * _D), lambda j: (j, 0)),
        out_shape=jax.ShapeDtypeStruct((_V, 2 * _D), jnp.float32),
    )(tabt)


def kernel(x, emb_table, W, b):
    xa = (x[:, :_HA].astype(jnp.int32) * 2)
    xb = jnp.pad(x[:, _HA:].astype(jnp.int32) * 2, ((0, 0), (0, _HA - _HB)))
    tab2 = _transpose_pad(emb_table.T).reshape(2 * _V, _D)
    pooled2 = _emb_pool(xa, xb, tab2)
    w2 = jnp.zeros((2 * _D, 2 * _NCLS), jnp.float32)
    w2 = w2.at[:_D, :_NCLS].set(W.T).at[_D:, _NCLS:].set(W.T)
    b2 = jnp.concatenate([b, b]).reshape(1, 2 * _NCLS)
    out2 = pl.pallas_call(
        _proj_body,
        out_shape=jax.ShapeDtypeStruct((_B // 2, 2 * _NCLS), jnp.float32),
    )(pooled2, w2, b2)
    return out2.reshape(_B, _NCLS)


# speedup vs baseline: 1.0020x; 1.0020x over previous
import functools

import jax
import jax.numpy as jnp
from jax import lax
from jax.experimental import pallas as pl
from jax.experimental.pallas import tpu as pltpu
from jax.experimental.pallas import tpu_sc as plsc

_NC = 2
_NS = 16
_NW = _NC * _NS
_L = 16

_B = 16384
_H = 200
_D = 64
_NCLS = 2

_HA = 128
_HB = _H - _HA
_KB = 4
_GIDX = _KB * _H
_BPW = _B // _NW
_NG = _BPW // _KB
_HALF = _NG // 2

_V = 1000000
_VBW = 128
_NFB = _V // _VBW
_FPW = _NFB // _NW
_TAIL = _V - _NFB * _VBW

_mesh = plsc.VectorSubcoreMesh(core_axis_name="c", subcore_axis_name="s")


@functools.partial(
    pl.kernel,
    mesh=_mesh,
    out_type=jax.ShapeDtypeStruct((_B // 2, 2 * _D), jnp.float32),
    scratch_types=[
        pltpu.VMEM((_KB, _HA), jnp.int32),
        pltpu.VMEM((_KB, _HA), jnp.int32),
        pltpu.VMEM((_KB, _HA), jnp.int32),
        pltpu.VMEM((_KB, _HA), jnp.int32),
        pltpu.VMEM((_GIDX, _D), jnp.float32),
        pltpu.VMEM((_GIDX, _D), jnp.float32),
        pltpu.VMEM((_HALF * _KB // 2, 2 * _D), jnp.float32),
        pltpu.SemaphoreType.DMA,
        pltpu.SemaphoreType.DMA,
        pltpu.SemaphoreType.DMA,
        pltpu.SemaphoreType.DMA,
    ],
    compiler_params=pltpu.CompilerParams(use_tc_tiling_on_sc=False),
)
def _emb_pool(xa_hbm, xb_hbm, tab_hbm, out_hbm,
              idxa_0, idxa_1, idxb_0, idxb_1, rows_a, rows_b, pooled_v,
              sem_ga, sem_gb, sem_ia, sem_ib):
    wid = lax.axis_index("s") * _NC + lax.axis_index("c")
    xrow0 = wid * _BPW
    neg = jnp.full((_L,), -jnp.inf, jnp.float32)
    idxa_bufs = (idxa_0, idxa_1)
    idxb_bufs = (idxb_0, idxb_1)
    rows_bufs = (rows_a, rows_b)
    sem_g = (sem_ga, sem_gb)
    sem_i = (sem_ia, sem_ib)

    def idx_copies(g, par):
        r0 = xrow0 + g * _KB
        return [
            pltpu.make_async_copy(
                xa_hbm.at[pl.ds(r0, _KB), :], idxa_bufs[par], sem_i[par]),
            pltpu.make_async_copy(
                xb_hbm.at[pl.ds(r0, _KB), :], idxb_bufs[par], sem_i[par]),
        ]

    def gather_copies(par):
        cps = []
        for e in range(_KB):
            cps.append(pltpu.make_async_copy(
                tab_hbm.at[idxa_bufs[par].at[e, pl.ds(0, _HA)]],
                rows_bufs[par].at[pl.ds(e * _H, _HA)], sem_g[par]))
            cps.append(pltpu.make_async_copy(
                tab_hbm.at[idxb_bufs[par].at[e, pl.ds(0, _HB)]],
                rows_bufs[par].at[pl.ds(e * _H + _HA, _HB)], sem_g[par]))
        return cps

    def compute(g, par):
        rows_v = rows_bufs[par]

        def rbody(i, acc):
            nxt = []
            for e in range(_KB):
                row = e * _H + 2 * i
                ae = []
                for c in range(_D // _L):
                    v0 = rows_v[row, pl.ds(c * _L, _L)]
                    v1 = rows_v[row + 1, pl.ds(c * _L, _L)]
                    ae.append(jnp.maximum(acc[e][c], jnp.maximum(v0, v1)))
                nxt.append(tuple(ae))
            return tuple(nxt)

        acc0 = tuple(tuple(neg for _ in range(_D // _L)) for _ in range(_KB))
        acc = lax.fori_loop(0, _H // 2, rbody, acc0)
        prow = (g & (_HALF - 1)) * (_KB // 2)
        for e in range(_KB):
            for c in range(_D // _L):
                col = (e % 2) * _D + c * _L
                pooled_v[prow + e // 2, pl.ds(col, _L)] = acc[e][c]

    for cp in idx_copies(0, 0):
        cp.start()
    for cp in idx_copies(0, 0):
        cp.wait()
    for cp in gather_copies(0):
        cp.start()
    for cp in idx_copies(1, 1):
        cp.start()

    def pair(gp, carry):
        for par in (0, 1):
            g = 2 * gp + par
            nxt = 1 - par
            for cp in idx_copies(g + 1, nxt):
                cp.wait()
            for cp in gather_copies(nxt):
                cp.start()
            for cp in gather_copies(par):
                cp.wait()
            for cp in idx_copies(g + 2, par):
                cp.start()
            compute(g, par)

            @pl.when(g == _HALF - 1)
            def _():
                pltpu.sync_copy(
                    pooled_v,
                    out_hbm.at[pl.ds(wid * (_BPW // 2), _HALF * _KB // 2), :])
        return carry

    lax.fori_loop(0, (_NG - 2) // 2, pair, 0)

    for cp in gather_copies(0):
        cp.wait()
    for cp in idx_copies(_NG - 1, 1):
        cp.wait()
    for cp in gather_copies(1):
        cp.start()
    compute(_NG - 2, 0)
    for cp in gather_copies(1):
        cp.wait()
    compute(_NG - 1, 1)
    pltpu.sync_copy(
        pooled_v,
        out_hbm.at[pl.ds(wid * (_BPW // 2) + _HALF * _KB // 2,
                         _HALF * _KB // 2), :])


def _proj_body(p_ref, w_ref, b_ref, o_ref):
    o_ref[:] = (jnp.dot(p_ref[:], w_ref[:], preferred_element_type=jnp.float32)
                + b_ref[:])


_TPB = 16384


def _tpad_body(t_ref, o_ref):
    t = jnp.swapaxes(t_ref[:], 0, 1)
    o_ref[:] = jnp.concatenate(
        [t, jnp.zeros((_TPB, _D), jnp.float32)], axis=1)


def _transpose_pad(tabt):
    grid = (_V + _TPB - 1) // _TPB
    return pl.pallas_call(
        _tpad_body,
        grid=(grid,),
        in_specs=[pl.BlockSpec((_D, _TPB), lambda j: (0, j))],
        out_specs=pl.BlockSpec((_TPB, 2 * _D), lambda j: (j, 0)),
        out_shape=jax.ShapeDtypeStruct((_V, 2 * _D), jnp.float32),
    )(tabt)


def kernel(x, emb_table, W, b):
    xa = (x[:, :_HA].astype(jnp.int32) * 2)
    xb = jnp.pad(x[:, _HA:].astype(jnp.int32) * 2, ((0, 0), (0, _HA - _HB)))
    tab2 = _transpose_pad(emb_table.T).reshape(2 * _V, _D)
    pooled2 = _emb_pool(xa, xb, tab2)
    w2 = jnp.zeros((2 * _D, 2 * _NCLS), jnp.float32)
    w2 = w2.at[:_D, :_NCLS].set(W.T).at[_D:, _NCLS:].set(W.T)
    b2 = jnp.concatenate([b, b]).reshape(1, 2 * _NCLS)
    out2 = pl.pallas_call(
        _proj_body,
        out_shape=jax.ShapeDtypeStruct((_B // 2, 2 * _NCLS), jnp.float32),
    )(pooled2, w2, b2)
    return out2.reshape(_B, _NCLS)
